# baseline (speedup 1.0000x reference)
import functools

import jax
import jax.numpy as jnp
from jax import lax
from jax.experimental import pallas as pl
from jax.experimental.pallas import tpu as pltpu
from jax.experimental.pallas import tpu_sc as plsc

N = 16384
D = 128
NSUBJ = 64

_NC = 2
_NS = 16
_NW = _NC * _NS
_BPW = N // _NW
_L = 16

_sc_mesh = plsc.VectorSubcoreMesh(core_axis_name="c", subcore_axis_name="s")


@functools.partial(
    pl.kernel,
    out_type=jax.ShapeDtypeStruct((N, 2), jnp.float32),
    mesh=_sc_mesh,
    scratch_types=[
        pltpu.VMEM((NSUBJ, 128), jnp.float32),
        pltpu.VMEM((_BPW,), jnp.int32),
        pltpu.VMEM((_BPW, 2), jnp.float32),
        pltpu.VMEM((_BPW, 2), jnp.float32),
    ],
    compiler_params=pltpu.CompilerParams(
        use_tc_tiling_on_sc=False, needs_layout_passes=False
    ),
)
def _sc_calibrate(tab_hbm, y_hbm, sid_hbm, out_hbm, tab_v, idx_v, y_v, out_v):
    wid = lax.axis_index("s") * _NC + lax.axis_index("c")
    base = wid * _BPW
    pltpu.sync_copy(tab_hbm, tab_v)
    pltpu.sync_copy(sid_hbm.at[pl.ds(base, _BPW)], idx_v)
    pltpu.sync_copy(y_hbm.at[pl.ds(base, _BPW)], y_v)
    lanes = lax.iota(jnp.int32, _L)
    cols = [jnp.full((_L,), j, jnp.int32) for j in range(6)]
    for i in range(_BPW // _L):
        tok = lanes + (i * _L)
        sidv = idx_v[pl.ds(i * _L, _L)]
        y0 = plsc.load_gather(y_v, [tok, cols[0]])
        y1 = plsc.load_gather(y_v, [tok, cols[1]])
        g0 = plsc.load_gather(tab_v, [sidv, cols[0]])
        g1 = plsc.load_gather(tab_v, [sidv, cols[1]])
        g2 = plsc.load_gather(tab_v, [sidv, cols[2]])
        g3 = plsc.load_gather(tab_v, [sidv, cols[3]])
        g4 = plsc.load_gather(tab_v, [sidv, cols[4]])
        g5 = plsc.load_gather(tab_v, [sidv, cols[5]])
        out0 = y0 * g0 + y1 * g2 + g4
        out1 = y0 * g1 + y1 * g3 + g5
        plsc.store_scatter(out_v, [tok, cols[0]], out0)
        plsc.store_scatter(out_v, [tok, cols[1]], out1)
    pltpu.sync_copy(out_v, out_hbm.at[pl.ds(base, _BPW)])


_BT = 2048


def _tc_matmul(x_ref, w_ref, b_ref, y_ref):
    y = jnp.dot(x_ref[...], w_ref[...], preferred_element_type=jnp.float32)
    y_ref[...] = y + b_ref[...]


def kernel(x, subjectID, W_net, b_net, W_cal, b_cal):
    tab = jnp.zeros((NSUBJ, 128), jnp.float32)
    tab = tab.at[:, 0:4].set(W_cal.reshape(NSUBJ, 4))
    tab = tab.at[:, 4:6].set(b_cal)
    y = pl.pallas_call(
        _tc_matmul,
        grid=(N // _BT,),
        in_specs=[
            pl.BlockSpec((_BT, D), lambda i: (i, 0)),
            pl.BlockSpec((D, 2), lambda i: (0, 0)),
            pl.BlockSpec((1, 2), lambda i: (0, 0)),
        ],
        out_specs=pl.BlockSpec((_BT, 2), lambda i: (i, 0)),
        out_shape=jax.ShapeDtypeStruct((N, 2), jnp.float32),
    )(x, W_net, b_net.reshape(1, 2))
    out = _sc_calibrate(tab, y, subjectID.astype(jnp.int32))
    return out


# speedup vs baseline: 1.3540x; 1.3540x over previous
import functools

import jax
import jax.numpy as jnp
from jax import lax
from jax.experimental import pallas as pl
from jax.experimental.pallas import tpu as pltpu
from jax.experimental.pallas import tpu_sc as plsc

N = 16384
D = 128
NSUBJ = 64
PD = 16
_PACK = 128 // PD
NG = N // _PACK

_NC = 2
_NS = 16
_NW = _NC * _NS
_BPW = N // _NW
_L = 16

_sc_mesh = plsc.VectorSubcoreMesh(core_axis_name="c", subcore_axis_name="s")


@functools.partial(
    pl.kernel,
    out_type=jax.ShapeDtypeStruct((NG, 128), jnp.float32),
    mesh=_sc_mesh,
    scratch_types=[
        pltpu.VMEM((NSUBJ, 128), jnp.float32),
        pltpu.VMEM((_BPW,), jnp.int32),
        pltpu.VMEM((256, 32), jnp.float32),
    ],
    compiler_params=pltpu.CompilerParams(
        use_tc_tiling_on_sc=False, needs_layout_passes=False
    ),
)
def _sc_gather(tab_hbm, sid_hbm, out_hbm, tab_v, idx_v, out_v):
    wid = lax.axis_index("s") * _NC + lax.axis_index("c")
    iblk = wid // 4
    q = wid % 4
    pltpu.sync_copy(tab_hbm, tab_v)
    pltpu.sync_copy(sid_hbm.at[pl.ds(wid * _BPW, _BPW)], idx_v)
    lanes = lax.iota(jnp.int32, _L)
    for g in range(_BPW // _L):
        sidv = idx_v[pl.ds(g * _L, _L)]
        rows = lanes + (16 * (g % 16))
        cbase = 16 * (g // 16)
        for j in range(6):
            colj = jnp.full((_L,), cbase + j, jnp.int32)
            vals = plsc.load_gather(tab_v, [sidv, jnp.full((_L,), j, jnp.int32)])
            plsc.store_scatter(out_v, [rows, colj], vals)
    pltpu.sync_copy(out_v, out_hbm.at[pl.ds(iblk * 256, 256), pl.ds(q * 32, 32)])


_BT = 2048
_PB = _BT // _PACK


def _selectors():
    r2 = lax.broadcasted_iota(jnp.int32, (2, PD), 0)
    c2 = lax.broadcasted_iota(jnp.int32, (2, PD), 1)
    s_sel = (c2 // 2 == r2).astype(jnp.float32)
    c1 = lax.broadcasted_iota(jnp.int32, (1, PD), 1)
    c_sel = (c1 // 2 == 2).astype(jnp.float32)
    rj = lax.broadcasted_iota(jnp.int32, (PD, 2), 0)
    ck = lax.broadcasted_iota(jnp.int32, (PD, 2), 1)
    r_sel = ((rj % 2 == ck) & (rj < 6)).astype(jnp.float32)
    return s_sel, c_sel, r_sel


def _tc_body(x_ref, g_ref, w_ref, b_ref, o_ref):
    s_sel, c_sel, r_sel = _selectors()
    y = jnp.dot(x_ref[...], w_ref[...], preferred_element_type=jnp.float32)
    y = y + b_ref[...]
    ybig = jnp.dot(y, s_sel, preferred_element_type=jnp.float32) + c_sel
    gfull = g_ref[...]
    for p in range(_PACK):
        gp = gfull[:, PD * p : PD * (p + 1)]
        tp = gp * ybig[_PB * p : _PB * (p + 1), :]
        o_ref[pl.ds(_PB * p, _PB), :] = jnp.dot(
            tp, r_sel, preferred_element_type=jnp.float32
        )


def kernel(x, subjectID, W_net, b_net, W_cal, b_cal):
    tab = jnp.zeros((NSUBJ, 128), jnp.float32)
    tab = tab.at[:, 0:4].set(W_cal.reshape(NSUBJ, 4))
    tab = tab.at[:, 4:6].set(b_cal)
    g = _sc_gather(tab, subjectID.astype(jnp.int32))
    out = pl.pallas_call(
        _tc_body,
        grid=(N // _BT,),
        in_specs=[
            pl.BlockSpec((_BT, D), lambda i: (i, 0)),
            pl.BlockSpec((_BT // _PACK, 128), lambda i: (i, 0)),
            pl.BlockSpec((D, 2), lambda i: (0, 0)),
            pl.BlockSpec((1, 2), lambda i: (0, 0)),
        ],
        out_specs=pl.BlockSpec((_BT, 2), lambda i: (i, 0)),
        out_shape=jax.ShapeDtypeStruct((N, 2), jnp.float32),
    )(x, g, W_net, b_net.reshape(1, 2))
    return out


# speedup vs baseline: 1.4754x; 1.0897x over previous
import functools

import jax
import jax.numpy as jnp
from jax import lax
from jax.experimental import pallas as pl
from jax.experimental.pallas import tpu as pltpu
from jax.experimental.pallas import tpu_sc as plsc

N = 16384
D = 128
NSUBJ = 64
PD = 16
_PACK = 128 // PD
NG = N // _PACK
_BT = 2048
_PB = _BT // _PACK

_NC = 2
_NS = 16
_NW = _NC * _NS
_BPW = N // _NW
_L = 16

_sc_mesh = plsc.VectorSubcoreMesh(core_axis_name="c", subcore_axis_name="s")


@functools.partial(
    pl.kernel,
    out_type=jax.ShapeDtypeStruct((NG, 128), jnp.float32),
    mesh=_sc_mesh,
    scratch_types=[
        pltpu.VMEM((NSUBJ, 128), jnp.float32),
        pltpu.VMEM((_BPW,), jnp.int32),
        pltpu.VMEM((256, 32), jnp.float32),
    ],
    compiler_params=pltpu.CompilerParams(
        use_tc_tiling_on_sc=False, needs_layout_passes=False
    ),
)
def _sc_gather(tab_hbm, sid_hbm, out_hbm, tab_v, idx_v, out_v):
    wid = lax.axis_index("s") * _NC + lax.axis_index("c")
    iblk = wid // 4
    q = wid % 4
    pltpu.sync_copy(tab_hbm, tab_v)
    pltpu.sync_copy(sid_hbm.at[pl.ds(wid * _BPW, _BPW)], idx_v)
    lanes = lax.iota(jnp.int32, _L)
    for g in range(_BPW // _L):
        sidv = idx_v[pl.ds(g * _L, _L)]
        rows = lanes + (16 * (g % 16))
        cbase = 16 * (g // 16)
        for j in range(6):
            colj = jnp.full((_L,), cbase + j, jnp.int32)
            vals = plsc.load_gather(tab_v, [sidv, jnp.full((_L,), j, jnp.int32)])
            plsc.store_scatter(out_v, [rows, colj], vals)
    pltpu.sync_copy(out_v, out_hbm.at[pl.ds(iblk * 256, 256), pl.ds(q * 32, 32)])


def _selectors():
    r2 = lax.broadcasted_iota(jnp.int32, (2, PD), 0)
    c2 = lax.broadcasted_iota(jnp.int32, (2, PD), 1)
    s_sel = (c2 // 2 == r2).astype(jnp.float32)
    c1 = lax.broadcasted_iota(jnp.int32, (1, PD), 1)
    c_sel = (c1 // 2 == 2).astype(jnp.float32)
    return s_sel, c_sel


def _tc_expand(x0, x1, x2, x3, x4, x5, x6, x7, w_ref, b_ref, y_ref):
    s_sel, c_sel = _selectors()
    w2 = jnp.dot(w_ref[...], s_sel, preferred_element_type=jnp.float32)
    cvec = jnp.dot(b_ref[...], s_sel, preferred_element_type=jnp.float32) + c_sel
    pieces = []
    for xp in (x0, x1, x2, x3, x4, x5, x6, x7):
        z = jnp.dot(xp[...], w2, preferred_element_type=jnp.float32) + cvec
        pieces.append(z)
    y_ref[...] = jnp.concatenate(pieces, axis=1)


def _tc_combine(y_ref, g_ref, o_ref):
    rj = lax.broadcasted_iota(jnp.int32, (128, PD), 0)
    ck = lax.broadcasted_iota(jnp.int32, (128, PD), 1)
    rbig = ((rj // PD == ck // 2) & (rj % PD < 6) & (rj % 2 == ck % 2)).astype(
        jnp.float32
    )
    t = y_ref[...] * g_ref[...]
    op = jnp.dot(t, rbig, preferred_element_type=jnp.float32)
    for p in range(_PACK):
        o_ref[pl.ds(_PB * p, _PB), :] = op[:, 2 * p : 2 * p + 2]


def kernel(x, subjectID, W_net, b_net, W_cal, b_cal):
    tab = jnp.concatenate(
        [W_cal.reshape(NSUBJ, 4), b_cal, jnp.zeros((NSUBJ, 122), jnp.float32)],
        axis=1,
    )
    g = _sc_gather(tab, subjectID.astype(jnp.int32))
    ybig = pl.pallas_call(
        _tc_expand,
        grid=(N // _BT,),
        in_specs=[
            pl.BlockSpec((_PB, D), functools.partial(lambda p, i: (8 * i + p, 0), p))
            for p in range(_PACK)
        ]
        + [
            pl.BlockSpec((D, 2), lambda i: (0, 0)),
            pl.BlockSpec((1, 2), lambda i: (0, 0)),
        ],
        out_specs=pl.BlockSpec((_PB, 128), lambda i: (i, 0)),
        out_shape=jax.ShapeDtypeStruct((NG, 128), jnp.float32),
    )(x, x, x, x, x, x, x, x, W_net, b_net.reshape(1, 2))
    out = pl.pallas_call(
        _tc_combine,
        grid=(N // _BT,),
        in_specs=[
            pl.BlockSpec((_PB, 128), lambda i: (i, 0)),
            pl.BlockSpec((_PB, 128), lambda i: (i, 0)),
        ],
        out_specs=pl.BlockSpec((_BT, 2), lambda i: (i, 0)),
        out_shape=jax.ShapeDtypeStruct((N, 2), jnp.float32),
    )(ybig, g)
    return out


# speedup vs baseline: 1.7119x; 1.1603x over previous
import functools

import jax
import jax.numpy as jnp
from jax import lax
from jax.experimental import pallas as pl
from jax.experimental.pallas import tpu as pltpu
from jax.experimental.pallas import tpu_sc as plsc

N = 16384
D = 128
NSUBJ = 64
PD = 16
_PACK = 128 // PD
NG = N // _PACK
_BT = 2048
_PB = _BT // _PACK

_NC = 2
_NS = 16
_NW = _NC * _NS
_BPW = N // _NW
_L = 16

_sc_mesh = plsc.VectorSubcoreMesh(core_axis_name="c", subcore_axis_name="s")


@functools.partial(
    pl.kernel,
    out_type=jax.ShapeDtypeStruct((NG, 128), jnp.float32),
    mesh=_sc_mesh,
    scratch_types=[
        pltpu.VMEM((NSUBJ, 128), jnp.float32),
        pltpu.VMEM((_BPW,), jnp.int32),
        pltpu.VMEM((256, 32), jnp.float32),
    ],
    compiler_params=pltpu.CompilerParams(
        use_tc_tiling_on_sc=False, needs_layout_passes=False
    ),
)
def _sc_gather(tab_hbm, sid_hbm, out_hbm, tab_v, idx_v, out_v):
    wid = lax.axis_index("s") * _NC + lax.axis_index("c")
    iblk = wid // 4
    q = wid % 4
    pltpu.sync_copy(tab_hbm, tab_v)
    pltpu.sync_copy(sid_hbm.at[pl.ds(wid * _BPW, _BPW)], idx_v)
    lanes = lax.iota(jnp.int32, _L)

    def body(g, carry):
        sidv = idx_v[pl.ds(g * _L, _L)]
        rows = lanes + (g % 16) * _L
        cbase = (g // 16) * _L
        for j in range(6):
            colj = jnp.full((_L,), j, jnp.int32) + cbase
            vals = plsc.load_gather(tab_v, [sidv, jnp.full((_L,), j, jnp.int32)])
            plsc.store_scatter(out_v, [rows, colj], vals)
        return carry

    lax.fori_loop(0, _BPW // _L, body, 0)
    pltpu.sync_copy(out_v, out_hbm.at[pl.ds(iblk * 256, 256), pl.ds(q * 32, 32)])


def _selectors():
    r2 = lax.broadcasted_iota(jnp.int32, (2, PD), 0)
    c2 = lax.broadcasted_iota(jnp.int32, (2, PD), 1)
    s_sel = (c2 // 2 == r2).astype(jnp.float32)
    c1 = lax.broadcasted_iota(jnp.int32, (1, PD), 1)
    c_sel = (c1 // 2 == 2).astype(jnp.float32)
    return s_sel, c_sel


def _tc_expand(x0, x1, x2, x3, x4, x5, x6, x7, w_ref, b_ref, y_ref):
    s_sel, c_sel = _selectors()
    w2 = jnp.dot(w_ref[...], s_sel, preferred_element_type=jnp.float32)
    cvec = jnp.dot(b_ref[...], s_sel, preferred_element_type=jnp.float32) + c_sel
    pieces = []
    for xp in (x0, x1, x2, x3, x4, x5, x6, x7):
        z = jnp.dot(xp[...], w2, preferred_element_type=jnp.float32) + cvec
        pieces.append(z)
    y_ref[...] = jnp.concatenate(pieces, axis=1)


def _tc_combine(y_ref, g_ref, o_ref):
    rj = lax.broadcasted_iota(jnp.int32, (128, PD), 0)
    ck = lax.broadcasted_iota(jnp.int32, (128, PD), 1)
    rbig = ((rj // PD == ck // 2) & (rj % PD < 6) & (rj % 2 == ck % 2)).astype(
        jnp.float32
    )
    t = y_ref[...] * g_ref[...]
    o_ref[...] = jnp.dot(t, rbig, preferred_element_type=jnp.float32)


def kernel(x, subjectID, W_net, b_net, W_cal, b_cal):
    tab = jnp.concatenate(
        [W_cal.reshape(NSUBJ, 4), b_cal, jnp.zeros((NSUBJ, 122), jnp.float32)],
        axis=1,
    )
    g = _sc_gather(tab, subjectID.astype(jnp.int32))
    ybig = pl.pallas_call(
        _tc_expand,
        grid=(N // _BT,),
        in_specs=[
            pl.BlockSpec((_PB, D), functools.partial(lambda p, i: (8 * i + p, 0), p))
            for p in range(_PACK)
        ]
        + [
            pl.BlockSpec((D, 2), lambda i: (0, 0)),
            pl.BlockSpec((1, 2), lambda i: (0, 0)),
        ],
        out_specs=pl.BlockSpec((_PB, 128), lambda i: (i, 0)),
        out_shape=jax.ShapeDtypeStruct((NG, 128), jnp.float32),
    )(x, x, x, x, x, x, x, x, W_net, b_net.reshape(1, 2))
    op = pl.pallas_call(
        _tc_combine,
        grid=(N // _BT,),
        in_specs=[
            pl.BlockSpec((_PB, 128), lambda i: (i, 0)),
            pl.BlockSpec((_PB, 128), lambda i: (i, 0)),
        ],
        out_specs=pl.BlockSpec((_PB, PD), lambda i: (i, 0)),
        out_shape=jax.ShapeDtypeStruct((NG, PD), jnp.float32),
    )(ybig, g)
    out = op.reshape(8, _PB, _PACK, 2).transpose(0, 2, 1, 3).reshape(N, 2)
    return out
